# Initial kernel scaffold; baseline (speedup 1.0000x reference)
import functools

import jax
import jax.numpy as jnp
from jax import lax
from jax.experimental import pallas as pl
from jax.experimental.pallas import tpu as pltpu
from jax.experimental.pallas import tpu_sc as plsc

N = 10000
E = 320000
F = 128
H = 64
D = 80
NC = 2
NS = 16
L = 16
NW = NC * NS
EPT = E // NW
EC = 80
NCH = EPT // EC
RPT = N // NS
ZR = 125


def _proj_body(x_ref, w_ref, al_ref, ar_ref, feat_ref, el_ref, er_ref):
    feat = jnp.dot(x_ref[...], w_ref[...], preferred_element_type=jnp.float32)
    feat_ref[...] = feat
    el_ref[...] = jnp.dot(feat, al_ref[...], preferred_element_type=jnp.float32)
    er_ref[...] = jnp.dot(feat, ar_ref[...], preferred_element_type=jnp.float32)


_proj = pl.pallas_call(
    _proj_body,
    out_shape=(
        jax.ShapeDtypeStruct((N, H), jnp.float32),
        jax.ShapeDtypeStruct((N, 1), jnp.float32),
        jax.ShapeDtypeStruct((N, 1), jnp.float32),
    ),
)


def _sc_body(src_hbm, dst_hbm, el_hbm, er_hbm, feat_hbm, out_hbm,
             src_b, dst_b, el_b, er_b, row_b, scat_b, zero_b, s_sh):
    c = lax.axis_index("c")
    s = lax.axis_index("s")
    wid = c * NS + s

    pltpu.sync_copy(src_hbm.at[wid], src_b)
    pltpu.sync_copy(dst_hbm.at[wid], dst_b)
    pltpu.sync_copy(el_hbm, el_b)
    pltpu.sync_copy(er_hbm, er_b)

    zv = jnp.zeros((L,), jnp.float32)

    def _zero_rows(ref, nrows):
        def body(r, _):
            for kk in range(D // L):
                ref[r, pl.ds(kk * L, L)] = zv
            return 0
        lax.fori_loop(0, nrows, body, 0)

    _zero_rows(zero_b, ZR)
    _zero_rows(scat_b, EC)

    def mbody(i, carry):
        mel, mer = carry
        mel = jnp.maximum(mel, el_b[pl.ds(i * L, L)])
        mer = jnp.maximum(mer, er_b[pl.ds(i * L, L)])
        return mel, mer
    neg = jnp.full((L,), -3.0e38, jnp.float32)
    mel, mer = lax.fori_loop(0, N // L, mbody, (neg, neg))
    msum = jnp.max(mel) + jnp.max(mer)
    m_sh = jnp.where(msum > 0, msum, 0.2 * msum)

    for z in range(RPT // ZR):
        pltpu.sync_copy(zero_b, s_sh.at[pl.ds(s * RPT + z * ZR, ZR)])
    plsc.subcore_barrier()

    iota = lax.iota(jnp.int32, L)
    colh = jnp.full((L,), H, jnp.int32)

    def chunk(j, _):
        pltpu.sync_copy(feat_hbm.at[src_b.at[j]], row_b)
        for k in range(EC // L):
            srcv = src_b[j, pl.ds(k * L, L)]
            dstv = dst_b[j, pl.ds(k * L, L)]
            x = plsc.load_gather(el_b, [srcv]) + plsc.load_gather(er_b, [dstv])
            e = jnp.where(x >= 0, x, 0.2 * x)
            a = jnp.exp(e - m_sh)
            plsc.store_scatter(scat_b, [iota + k * L, colh], a)
        def edge(t, _):
            a_s = scat_b[t, H]
            for cc in range(H // L):
                scat_b[t, pl.ds(cc * L, L)] = row_b[t, pl.ds(cc * L, L)] * a_s
            return 0
        lax.fori_loop(0, EC, edge, 0)
        pltpu.sync_copy(scat_b, s_sh.at[dst_b.at[j]], add=True)
        return 0

    lax.fori_loop(0, NCH, chunk, 0)
    plsc.subcore_barrier()

    for z in range(RPT // ZR):
        base = s * RPT + z * ZR
        pltpu.sync_copy(s_sh.at[pl.ds(base, ZR)], zero_b)
        pltpu.sync_copy(zero_b, out_hbm.at[c, pl.ds(base, ZR)])


_sc_gat = functools.partial(
    pl.kernel,
    out_type=jax.ShapeDtypeStruct((NC, N, D), jnp.float32),
    mesh=plsc.VectorSubcoreMesh(core_axis_name="c", subcore_axis_name="s"),
    scratch_types=[
        pltpu.VMEM((NCH, EC), jnp.int32),
        pltpu.VMEM((NCH, EC), jnp.int32),
        pltpu.VMEM((N,), jnp.float32),
        pltpu.VMEM((N,), jnp.float32),
        pltpu.VMEM((EC, H), jnp.float32),
        pltpu.VMEM((EC, D), jnp.float32),
        pltpu.VMEM((ZR, D), jnp.float32),
        pltpu.VMEM_SHARED((N, D), jnp.float32),
    ],
)(_sc_body)


def _head_body(s_ref, bias_ref, fcw_ref, fcb_ref, y_ref):
    num = s_ref[0, :, :H] + s_ref[1, :, :H]
    den = s_ref[0, :, H:H + 1] + s_ref[1, :, H:H + 1]
    rst = num / (den + 1e-16)
    h = rst + bias_ref[...]
    h = jnp.where(h > 0, h, jnp.exp(jnp.minimum(h, 0.0)) - 1.0)
    hg = jnp.mean(h, axis=0, keepdims=True)
    logit = jnp.sum(hg * fcw_ref[...], axis=1, keepdims=True) + fcb_ref[...]
    y_ref[...] = 1.0 / (1.0 + jnp.exp(-logit))


_head = pl.pallas_call(
    _head_body,
    out_shape=jax.ShapeDtypeStruct((1, 1), jnp.float32),
)


def kernel(features, edge_index, W, attn_l, attn_r, bias, fc_W, fc_b):
    feat, el, er = _proj(features, W,
                         attn_l.reshape(H, 1), attn_r.reshape(H, 1))
    src3 = edge_index[0].reshape(NW, NCH, EC)
    dst3 = edge_index[1].reshape(NW, NCH, EC)
    partials = _sc_gat(src3, dst3, el.reshape(N), er.reshape(N), feat)
    return _head(partials, bias.reshape(1, H), fc_W, fc_b.reshape(1, 1))


# speedup vs baseline: 17.1246x; 17.1246x over previous
import functools

import jax
import jax.numpy as jnp
from jax import lax
from jax.experimental import pallas as pl
from jax.experimental.pallas import tpu as pltpu
from jax.experimental.pallas import tpu_sc as plsc

N = 10000
E = 320000
F = 128
H = 64
D = 128
NC = 2
NS = 16
L = 16
NW = NC * NS
EC = 80
SB = 8
NSB = 16
NCH = SB * NSB
EPT = NCH * EC
EPAD = NW * EPT
NP = 10240
RPT = NP // NS
WC = 8


def _proj_body(x_ref, w_ref, al_ref, ar_ref, feat_ref, er_ref, m_ref):
    feat = jnp.dot(x_ref[...], w_ref[...], preferred_element_type=jnp.float32)
    feat_ref[...] = feat
    el = jnp.dot(feat, al_ref[...], preferred_element_type=jnp.float32)
    er = jnp.dot(feat, ar_ref[...], preferred_element_type=jnp.float32)
    er_ref[...] = er
    msum = jnp.max(el) + jnp.max(er)
    m = jnp.where(msum > 0, msum, 0.2 * msum)
    m_ref[...] = jnp.full((1, L), m, jnp.float32)


_proj = pl.pallas_call(
    _proj_body,
    out_shape=(
        jax.ShapeDtypeStruct((N, D), jnp.float32),
        jax.ShapeDtypeStruct((N, 1), jnp.float32),
        jax.ShapeDtypeStruct((1, L), jnp.float32),
    ),
)


def _sc_body(src_hbm, dst_hbm, er_hbm, m_hbm, feat_hbm, out_hbm,
             src_sb, dst_sb, er_b, m_b, row_b, scat_b, s_sh):
    c = lax.axis_index("c")
    s = lax.axis_index("s")
    wid = c * NS + s

    pltpu.sync_copy(er_hbm, er_b)
    pltpu.sync_copy(m_hbm, m_b)

    zv = jnp.zeros((L,), jnp.float32)

    def zrow(r, _):
        for kk in range(D // L):
            scat_b[r, pl.ds(kk * L, L)] = zv
        return 0
    lax.fori_loop(0, EC, zrow, 0)

    m_sh = m_b[...]

    for z in range(WC):
        pltpu.sync_copy(scat_b, s_sh.at[pl.ds(s * RPT + z * EC, EC)])
    plsc.subcore_barrier()

    iota = lax.iota(jnp.int32, L)
    colh = jnp.full((L,), H, jnp.int32)

    def super_chunk(jj, _):
        pltpu.sync_copy(src_hbm.at[wid, pl.ds(jj * SB, SB)], src_sb)
        pltpu.sync_copy(dst_hbm.at[wid, pl.ds(jj * SB, SB)], dst_sb)

        def chunk(j, _):
            pltpu.sync_copy(feat_hbm.at[src_sb.at[j]], row_b)
            for k in range(EC // L):
                rows = iota + k * L
                elv = plsc.load_gather(row_b, [rows, colh])
                erv = plsc.load_gather(er_b, [dst_sb[j, pl.ds(k * L, L)]])
                x = elv + erv
                e = jnp.where(x >= 0, x, 0.2 * x)
                a = jnp.exp(e - m_sh)
                plsc.store_scatter(scat_b, [rows, colh], a)
            def edge(t, _):
                a_s = scat_b[t, pl.ds(H, L)][0]
                for cc in range(H // L):
                    scat_b[t, pl.ds(cc * L, L)] = row_b[t, pl.ds(cc * L, L)] * a_s
                return 0
            lax.fori_loop(0, EC, edge, 0)
            pltpu.sync_copy(scat_b, s_sh.at[dst_sb.at[j]], add=True)
            return 0

        lax.fori_loop(0, SB, chunk, 0)
        return 0

    lax.fori_loop(0, NSB, super_chunk, 0)
    plsc.subcore_barrier()

    for z in range(WC):
        base = s * RPT + z * EC
        pltpu.sync_copy(s_sh.at[pl.ds(base, EC)], scat_b)
        pltpu.sync_copy(scat_b, out_hbm.at[c, pl.ds(base, EC)])


_sc_gat = functools.partial(
    pl.kernel,
    out_type=jax.ShapeDtypeStruct((NC, NP, D), jnp.float32),
    mesh=plsc.VectorSubcoreMesh(core_axis_name="c", subcore_axis_name="s"),
    compiler_params=pltpu.CompilerParams(needs_layout_passes=False),
    scratch_types=[
        pltpu.VMEM((SB, EC), jnp.int32),
        pltpu.VMEM((SB, EC), jnp.int32),
        pltpu.VMEM((NP,), jnp.float32),
        pltpu.VMEM((L,), jnp.float32),
        pltpu.VMEM((EC, D), jnp.float32),
        pltpu.VMEM((EC, D), jnp.float32),
        pltpu.VMEM_SHARED((NP, D), jnp.float32),
    ],
)(_sc_body)


def _head_body(s_ref, bias_ref, fcw_ref, fcb_ref, y_ref):
    num = s_ref[0, :N, :H] + s_ref[1, :N, :H]
    den = s_ref[0, :N, H:H + 1] + s_ref[1, :N, H:H + 1]
    rst = num / (den + 1e-16)
    h = rst + bias_ref[...]
    h = jnp.where(h > 0, h, jnp.exp(jnp.minimum(h, 0.0)) - 1.0)
    hg = jnp.mean(h, axis=0, keepdims=True)
    logit = jnp.sum(hg * fcw_ref[...], axis=1, keepdims=True) + fcb_ref[...]
    y_ref[...] = 1.0 / (1.0 + jnp.exp(-logit))


_head = pl.pallas_call(
    _head_body,
    out_shape=jax.ShapeDtypeStruct((1, 1), jnp.float32),
)


def kernel(features, edge_index, W, attn_l, attn_r, bias, fc_W, fc_b):
    al = attn_l.reshape(H)
    ar = attn_r.reshape(H)
    Wp = jnp.concatenate(
        [W, (W @ al)[:, None], jnp.zeros((F, D - H - 1), jnp.float32)], axis=1)
    alp = jnp.pad(al[:, None], ((0, D - H), (0, 0)))
    arp = jnp.pad(ar[:, None], ((0, D - H), (0, 0)))
    feat, er, m = _proj(features, Wp, alp, arp)
    pad = EPAD - E
    src_p = jnp.concatenate([edge_index[0], jnp.zeros((pad,), jnp.int32)])
    dst_p = jnp.concatenate([edge_index[1], jnp.full((pad,), N, jnp.int32)])
    er_p = jnp.pad(er.reshape(N), (0, NP - N))
    partials = _sc_gat(src_p.reshape(NW, NCH, EC), dst_p.reshape(NW, NCH, EC),
                       er_p, m.reshape(L), feat)
    return _head(partials, bias.reshape(1, H), fc_W, fc_b.reshape(1, 1))


# speedup vs baseline: 20.7637x; 1.2125x over previous
import functools

import jax
import jax.numpy as jnp
from jax import lax
from jax.experimental import pallas as pl
from jax.experimental.pallas import tpu as pltpu
from jax.experimental.pallas import tpu_sc as plsc

N = 10000
E = 320000
F = 128
H = 64
D = 128
NC = 2
NS = 16
L = 16
NW = NC * NS
EC = 64
SB = 8
NSB = 20
NCH = SB * NSB
EPT = NCH * EC
EPAD = NW * EPT
NP = 10240
RPT = NP // NS
WC = RPT // EC


def _proj_body(x_ref, w_ref, al_ref, ar_ref, feat_ref, er_ref, m_ref):
    feat = jnp.dot(x_ref[...], w_ref[...], preferred_element_type=jnp.float32)
    feat_ref[...] = feat
    el = jnp.dot(feat, al_ref[...], preferred_element_type=jnp.float32)
    er = jnp.dot(feat, ar_ref[...], preferred_element_type=jnp.float32)
    er_ref[...] = er
    msum = jnp.max(el) + jnp.max(er)
    m = jnp.where(msum > 0, msum, 0.2 * msum)
    m_ref[...] = jnp.full((1, L), m, jnp.float32)


_proj = pl.pallas_call(
    _proj_body,
    out_shape=(
        jax.ShapeDtypeStruct((N, D), jnp.float32),
        jax.ShapeDtypeStruct((N, 1), jnp.float32),
        jax.ShapeDtypeStruct((1, L), jnp.float32),
    ),
)


def _sc_body(src_hbm, dst_hbm, er_hbm, m_hbm, feat_hbm, out_hbm,
             src0, dst0, src1, dst1, er_b, m_b, row0, row1, sc0, sc1,
             s_sh, gs0, gs1, ss0, ss1):
    c = lax.axis_index("c")
    s = lax.axis_index("s")
    wid = c * NS + s

    pltpu.sync_copy(er_hbm, er_b)
    pltpu.sync_copy(m_hbm, m_b)

    zv = jnp.zeros((L,), jnp.float32)

    def zrow(r, _):
        for kk in range(D // L):
            sc0[r, pl.ds(kk * L, L)] = zv
            sc1[r, pl.ds(kk * L, L)] = zv
        return 0
    lax.fori_loop(0, EC, zrow, 0)

    m_sh = m_b[...]

    for z in range(WC):
        pltpu.sync_copy(sc0, s_sh.at[pl.ds(s * RPT + z * EC, EC)])

    pltpu.sync_copy(src_hbm.at[wid, pl.ds(0, SB)], src0)
    pltpu.sync_copy(dst_hbm.at[wid, pl.ds(0, SB)], dst0)
    pltpu.async_copy(feat_hbm.at[src0.at[0]], row0, gs0)
    plsc.subcore_barrier()

    iota = lax.iota(jnp.int32, L)
    colh = jnp.full((L,), H, jnp.int32)
    rows_ = (row0, row1)
    scs_ = (sc0, sc1)
    gss_ = (gs0, gs1)
    sss_ = (ss0, ss1)
    sbs_ = ((src0, dst0), (src1, dst1))

    def super_pair(jj2, _):
        for ii in range(2):
            jj = jj2 * 2 + ii
            src_c, dst_c = sbs_[ii]
            src_o, dst_o = sbs_[1 - ii]

            def pair(g2, _):
                for b in range(2):
                    j = g2 * 2 + b
                    rb, scb, gsem, ssem = rows_[b], scs_[b], gss_[b], sss_[b]
                    ro, gso = rows_[1 - b], gss_[1 - b]
                    pltpu.make_async_copy(feat_hbm.at[src_c.at[j]], rb, gsem).wait()
                    if b == 0:
                        pltpu.async_copy(feat_hbm.at[src_c.at[j + 1]], ro, gso)
                    else:
                        @pl.when(g2 < SB // 2 - 1)
                        def _nxt():
                            pltpu.async_copy(feat_hbm.at[src_c.at[j + 1]], ro, gso)

                        @pl.when((g2 == SB // 2 - 1) & (jj < NSB - 1))
                        def _nxt_super():
                            pltpu.sync_copy(
                                src_hbm.at[wid, pl.ds((jj + 1) * SB, SB)], src_o)
                            pltpu.sync_copy(
                                dst_hbm.at[wid, pl.ds((jj + 1) * SB, SB)], dst_o)
                            pltpu.async_copy(feat_hbm.at[src_o.at[0]], ro, gso)
                    if ii == 1:
                        pltpu.make_async_copy(
                            scb, s_sh.at[dst_c.at[j]], ssem).wait()
                    else:
                        @pl.when((jj2 > 0) | (g2 > 0))
                        def _drain():
                            pltpu.make_async_copy(
                                scb, s_sh.at[dst_c.at[j]], ssem).wait()
                    for k in range(EC // L):
                        rws = iota + k * L
                        elv = plsc.load_gather(rb, [rws, colh])
                        erv = plsc.load_gather(er_b, [dst_c[j, pl.ds(k * L, L)]])
                        x = elv + erv
                        e = jnp.where(x >= 0, x, 0.2 * x)
                        a = jnp.exp(e - m_sh)
                        plsc.store_scatter(scb, [rws, colh], a)
                    def edge(t, _):
                        a_s = scb[t, pl.ds(H, L)][0]
                        for cc in range(H // L):
                            scb[t, pl.ds(cc * L, L)] = rb[t, pl.ds(cc * L, L)] * a_s
                        return 0
                    lax.fori_loop(0, EC, edge, 0)
                    pltpu.async_copy(scb, s_sh.at[dst_c.at[j]], ssem, add=True)
                return 0

            lax.fori_loop(0, SB // 2, pair, 0)
        return 0

    lax.fori_loop(0, NSB // 2, super_pair, 0)
    pltpu.make_async_copy(sc0, s_sh.at[dst1.at[SB - 2]], ss0).wait()
    pltpu.make_async_copy(sc1, s_sh.at[dst1.at[SB - 1]], ss1).wait()
    plsc.subcore_barrier()

    for z in range(WC):
        base = s * RPT + z * EC
        pltpu.sync_copy(s_sh.at[pl.ds(base, EC)], sc0)
        pltpu.sync_copy(sc0, out_hbm.at[c, pl.ds(base, EC)])


_sc_gat = functools.partial(
    pl.kernel,
    out_type=jax.ShapeDtypeStruct((NC, NP, D), jnp.float32),
    mesh=plsc.VectorSubcoreMesh(core_axis_name="c", subcore_axis_name="s"),
    compiler_params=pltpu.CompilerParams(needs_layout_passes=False),
    scratch_types=[
        pltpu.VMEM((SB, EC), jnp.int32),
        pltpu.VMEM((SB, EC), jnp.int32),
        pltpu.VMEM((SB, EC), jnp.int32),
        pltpu.VMEM((SB, EC), jnp.int32),
        pltpu.VMEM((NP,), jnp.float32),
        pltpu.VMEM((L,), jnp.float32),
        pltpu.VMEM((EC, D), jnp.float32),
        pltpu.VMEM((EC, D), jnp.float32),
        pltpu.VMEM((EC, D), jnp.float32),
        pltpu.VMEM((EC, D), jnp.float32),
        pltpu.VMEM_SHARED((NP, D), jnp.float32),
        pltpu.SemaphoreType.DMA,
        pltpu.SemaphoreType.DMA,
        pltpu.SemaphoreType.DMA,
        pltpu.SemaphoreType.DMA,
    ],
)(_sc_body)


def _head_body(s_ref, bias_ref, fcw_ref, fcb_ref, y_ref):
    num = s_ref[0, :N, :H] + s_ref[1, :N, :H]
    den = s_ref[0, :N, H:H + 1] + s_ref[1, :N, H:H + 1]
    rst = num / (den + 1e-16)
    h = rst + bias_ref[...]
    h = jnp.where(h > 0, h, jnp.exp(jnp.minimum(h, 0.0)) - 1.0)
    hg = jnp.mean(h, axis=0, keepdims=True)
    logit = jnp.sum(hg * fcw_ref[...], axis=1, keepdims=True) + fcb_ref[...]
    y_ref[...] = 1.0 / (1.0 + jnp.exp(-logit))


_head = pl.pallas_call(
    _head_body,
    out_shape=jax.ShapeDtypeStruct((1, 1), jnp.float32),
)


def kernel(features, edge_index, W, attn_l, attn_r, bias, fc_W, fc_b):
    al = attn_l.reshape(H)
    ar = attn_r.reshape(H)
    Wp = jnp.concatenate(
        [W, (W @ al)[:, None], jnp.zeros((F, D - H - 1), jnp.float32)], axis=1)
    alp = jnp.pad(al[:, None], ((0, D - H), (0, 0)))
    arp = jnp.pad(ar[:, None], ((0, D - H), (0, 0)))
    feat, er, m = _proj(features, Wp, alp, arp)
    pad = EPAD - E
    src_p = jnp.concatenate([edge_index[0], jnp.zeros((pad,), jnp.int32)])
    dst_p = jnp.concatenate([edge_index[1], jnp.full((pad,), N, jnp.int32)])
    er_p = jnp.pad(er.reshape(N), (0, NP - N))
    partials = _sc_gat(src_p.reshape(NW, NCH, EC), dst_p.reshape(NW, NCH, EC),
                       er_p, m.reshape(L), feat)
    return _head(partials, bias.reshape(1, H), fc_W, fc_b.reshape(1, 1))
